# baseline (speedup 1.0000x reference)
import functools

import jax
import jax.numpy as jnp
from jax import lax
from jax.experimental import pallas as pl
from jax.experimental.pallas import tpu as pltpu
from jax.experimental.pallas import tpu_sc as plsc

B = 16384
NC, NS, L = 2, 16, 16
NW = NC * NS
RPW = B // NW
IB = RPW // 128

_mesh = plsc.VectorSubcoreMesh(
    core_axis_name="c", subcore_axis_name="s", num_cores=NC, num_subcores=NS
)


@functools.partial(
    pl.kernel,
    out_type=(
        jax.ShapeDtypeStruct((16, B), jnp.float32),
        jax.ShapeDtypeStruct((8, B), jnp.float32),
        jax.ShapeDtypeStruct((8, B), jnp.float32),
    ),
    mesh=_mesh,
    compiler_params=pltpu.CompilerParams(
        use_tc_tiling_on_sc=False,
        needs_layout_passes=False,
        disable_bounds_checks=True,
        disable_semaphore_checks=True,
        skip_device_barrier=True,
    ),
    scratch_types=[
        pltpu.VMEM((IB, 128), jnp.int32),
        pltpu.VMEM((IB, 128), jnp.int32),
        pltpu.VMEM((IB, 128), jnp.int32),
        pltpu.VMEM((32, 128), jnp.float32),
        pltpu.VMEM((16, RPW), jnp.float32),
        pltpu.VMEM((8, RPW), jnp.float32),
        pltpu.VMEM((8, RPW), jnp.float32),
        pltpu.SemaphoreType.DMA,
        pltpu.SemaphoreType.DMA,
    ],
)
def _sc_embed(idx_hbm, wt_hbm,
              out_charT, out_roleT, out_buffT,
              idxc_v, idxr_v, idxb_v, wt_v, charT_v, roleT_v, buffT_v,
              sem_in, sem_out):
    wid = lax.axis_index("s") * NC + lax.axis_index("c")
    base = wid * RPW

    cps = [
        pltpu.async_copy(idx_hbm.at[0, pl.ds(wid * IB, IB)], idxc_v, sem_in),
        pltpu.async_copy(idx_hbm.at[1, pl.ds(wid * IB, IB)], idxr_v, sem_in),
        pltpu.async_copy(idx_hbm.at[2, pl.ds(wid * IB, IB)], idxb_v, sem_in),
        pltpu.async_copy(wt_hbm, wt_v, sem_in),
    ]
    for cp in cps:
        cp.wait()

    @plsc.parallel_loop(0, IB)
    def _row_block(j):
        @plsc.parallel_loop(0, 8, unroll=2)
        def _group(k):
            sl = pl.ds(k * L, L)
            ic = idxc_v[j, sl]
            ir = idxr_v[j, sl]
            ib = idxb_v[j, sl]
            out_sl = pl.ds(j * 128 + k * L, L)
            for c in range(16):
                cc = jnp.full((L,), c, jnp.int32)
                charT_v[c, out_sl] = plsc.load_gather(wt_v, [cc, ic])
            for c in range(8):
                cc = jnp.full((L,), 16 + c, jnp.int32)
                roleT_v[c, out_sl] = plsc.load_gather(wt_v, [cc, ir])
            for c in range(6):
                cc = jnp.full((L,), 24 + c, jnp.int32)
                buffT_v[c, out_sl] = plsc.load_gather(wt_v, [cc, ib])

    outs = [
        pltpu.async_copy(charT_v, out_charT.at[:, pl.ds(base, RPW)], sem_out),
        pltpu.async_copy(roleT_v, out_roleT.at[:, pl.ds(base, RPW)], sem_out),
        pltpu.async_copy(buffT_v, out_buffT.at[:, pl.ds(base, RPW)], sem_out),
    ]
    for cp in outs:
        cp.wait()


def kernel(x, W_char, W_role, W_buff):
    idx3 = x[:, 0:3].astype(jnp.int32).T.reshape(3, B // 128, 128)
    wt = jnp.concatenate([
        jnp.pad(W_char.T, ((0, 0), (0, 28))),
        jnp.pad(W_role.T, ((0, 0), (0, 120))),
        jnp.pad(W_buff.T, ((0, 2), (0, 78))),
    ], axis=0)
    charT, roleT, buffT = _sc_embed(idx3, wt)
    my_char = charT.T
    my_role = roleT.T
    my_buff = buffT[0:6, :].T
    my_states = x[:, 3:76]
    return (my_char, my_role, my_buff, my_states)


# speedup vs baseline: 1.0044x; 1.0044x over previous
import functools

import jax
import jax.numpy as jnp
from jax import lax
from jax.experimental import pallas as pl
from jax.experimental.pallas import tpu as pltpu
from jax.experimental.pallas import tpu_sc as plsc

B = 16384
NC, NS, L = 2, 16, 16
NW = NC * NS
RPW = B // NW
IB = RPW // 128

_mesh = plsc.VectorSubcoreMesh(
    core_axis_name="c", subcore_axis_name="s", num_cores=NC, num_subcores=NS
)


@functools.partial(
    pl.kernel,
    out_type=(
        jax.ShapeDtypeStruct((16, B), jnp.float32),
        jax.ShapeDtypeStruct((8, B), jnp.float32),
        jax.ShapeDtypeStruct((8, B), jnp.float32),
    ),
    mesh=_mesh,
    compiler_params=pltpu.CompilerParams(
        use_tc_tiling_on_sc=False,
        needs_layout_passes=False,
        disable_bounds_checks=True,
        disable_semaphore_checks=True,
        skip_device_barrier=True,
    ),
    scratch_types=[
        pltpu.VMEM((IB, 128), jnp.int32),
        pltpu.VMEM((IB, 128), jnp.int32),
        pltpu.VMEM((IB, 128), jnp.int32),
        pltpu.VMEM((32, 128), jnp.float32),
        pltpu.VMEM((16, RPW), jnp.float32),
        pltpu.VMEM((8, RPW), jnp.float32),
        pltpu.VMEM((8, RPW), jnp.float32),
        pltpu.SemaphoreType.DMA,
        pltpu.SemaphoreType.DMA,
    ],
)
def _sc_embed(idx_hbm, wt_hbm,
              out_charT, out_roleT, out_buffT,
              idxc_v, idxr_v, idxb_v, wt_v, charT_v, roleT_v, buffT_v,
              sem_in, sem_out):
    wid = lax.axis_index("s") * NC + lax.axis_index("c")
    base = wid * RPW

    cps = [
        pltpu.async_copy(idx_hbm.at[0, pl.ds(wid * IB, IB)], idxc_v, sem_in),
        pltpu.async_copy(idx_hbm.at[1, pl.ds(wid * IB, IB)], idxr_v, sem_in),
        pltpu.async_copy(idx_hbm.at[2, pl.ds(wid * IB, IB)], idxb_v, sem_in),
        pltpu.async_copy(wt_hbm, wt_v, sem_in),
    ]
    for cp in cps:
        cp.wait()

    @plsc.parallel_loop(0, IB)
    def _row_block(j):
        @plsc.parallel_loop(0, 8)
        def _group(k):
            sl = pl.ds(k * L, L)
            ic = idxc_v[j, sl]
            ir = idxr_v[j, sl]
            ib = idxb_v[j, sl]
            out_sl = pl.ds(j * 128 + k * L, L)
            for c in range(16):
                cc = jnp.full((L,), c, jnp.int32)
                charT_v[c, out_sl] = plsc.load_gather(wt_v, [cc, ic])
            for c in range(8):
                cc = jnp.full((L,), 16 + c, jnp.int32)
                roleT_v[c, out_sl] = plsc.load_gather(wt_v, [cc, ir])
            for c in range(6):
                cc = jnp.full((L,), 24 + c, jnp.int32)
                buffT_v[c, out_sl] = plsc.load_gather(wt_v, [cc, ib])

    outs = [
        pltpu.async_copy(charT_v, out_charT.at[:, pl.ds(base, RPW)], sem_out),
        pltpu.async_copy(roleT_v, out_roleT.at[:, pl.ds(base, RPW)], sem_out),
        pltpu.async_copy(buffT_v, out_buffT.at[:, pl.ds(base, RPW)], sem_out),
    ]
    for cp in outs:
        cp.wait()


def kernel(x, W_char, W_role, W_buff):
    idx3 = x[:, 0:3].astype(jnp.int32).T.reshape(3, B // 128, 128)
    wt = jnp.concatenate([
        jnp.pad(W_char.T, ((0, 0), (0, 28))),
        jnp.pad(W_role.T, ((0, 0), (0, 120))),
        jnp.pad(W_buff.T, ((0, 2), (0, 78))),
    ], axis=0)
    charT, roleT, buffT = _sc_embed(idx3, wt)
    my_char = charT.T
    my_role = roleT.T
    my_buff = buffT[0:6, :].T
    my_states = x[:, 3:76]
    return (my_char, my_role, my_buff, my_states)
